# baseline (speedup 1.0000x reference)
import jax
import jax.numpy as jnp
from jax import lax
from jax.experimental import pallas as pl
from jax.experimental.pallas import tpu as pltpu
from jax.experimental.pallas import tpu_sc as plsc

N = 10000
D = 128
E = 320000

NC = 2
NS = 16
B = 128
NBW = 160
NBT = NS * NBW
E_PAD = NBT * B
N_PAD = 10240
RPT = N_PAD // NS
IC = 16
NCH = NBW // IC

_mesh = plsc.VectorSubcoreMesh(core_axis_name="c", subcore_axis_name="s",
                               num_cores=NC, num_subcores=NS)


HB = B // 2


def _agg_body(table, srcb, dstb, summ_out, summ_acc,
              srcA, dstA, srcB, dstB, rows0, rows1,
              gl0, gl1, gh0, gh1, isemA, isemB):
    c = lax.axis_index("c")
    s = lax.axis_index("s")
    base = s * NBW
    r0 = s * RPT
    rows = (rows0, rows1)
    glsems = (gl0, gl1)
    ghsems = (gh0, gh1)

    def _issue_gather(idx_row, buf):
        pltpu.async_copy(table.at[idx_row.at[pl.ds(0, HB)]],
                         rows[buf].at[pl.ds(0, HB)], glsems[buf])
        pltpu.async_copy(table.at[idx_row.at[pl.ds(HB, HB)]],
                         rows[buf].at[pl.ds(HB, HB)], ghsems[buf])

    def _wait_gather(buf):
        pltpu.make_async_copy(table.at[pl.ds(0, HB)],
                              rows[buf].at[pl.ds(0, HB)], glsems[buf]).wait()
        pltpu.make_async_copy(table.at[pl.ds(0, HB)],
                              rows[buf].at[pl.ds(HB, HB)], ghsems[buf]).wait()

    def _fill_zrow(i, carry):
        for k in range(D // 16):
            z = jnp.zeros((16,), jnp.float32)
            rows0[i, pl.ds(k * 16, 16)] = z
            rows1[i, pl.ds(k * 16, 16)] = z
        return carry
    lax.fori_loop(0, B, _fill_zrow, 0)
    for k in range(RPT // B):
        pltpu.sync_copy(rows0, summ_acc.at[pl.ds(r0 + k * B, B)])
    pltpu.sync_copy(srcb.at[c, pl.ds(base, IC)], srcA)
    pltpu.sync_copy(dstb.at[c, pl.ds(base, IC)], dstA)
    _issue_gather(srcA.at[0], 0)
    plsc.subcore_barrier()

    def _do_chunk(ch, srcX, dstX, srcY, dstY, isemY, more):
        for j in range(IC):
            b = j % 2
            nb = (j + 1) % 2
            if j == 0:
                def _prefetch():
                    cb = base + (ch + 1) * IC
                    pltpu.async_copy(srcb.at[c, pl.ds(cb, IC)], srcY, isemY)
                    pltpu.async_copy(dstb.at[c, pl.ds(cb, IC)], dstY, isemY)
                if more is True:
                    _prefetch()
                else:
                    pl.when(more)(_prefetch)
            if j < IC - 1:
                _issue_gather(srcX.at[j + 1], nb)
            else:
                def _next_gather():
                    cb = base + (ch + 1) * IC
                    pltpu.make_async_copy(srcb.at[c, pl.ds(cb, IC)], srcY,
                                          isemY).wait()
                    pltpu.make_async_copy(dstb.at[c, pl.ds(cb, IC)], dstY,
                                          isemY).wait()
                    _issue_gather(srcY.at[0], nb)
                if more is True:
                    _next_gather()
                else:
                    pl.when(more)(_next_gather)
            _wait_gather(b)
            pltpu.sync_copy(rows[b], summ_acc.at[dstX.at[j]], add=True)

    def _pair(u, carry):
        _do_chunk(2 * u, srcA, dstA, srcB, dstB, isemB, True)
        _do_chunk(2 * u + 1, srcB, dstB, srcA, dstA, isemA, u + 1 < NCH // 2)
        return carry
    lax.fori_loop(0, NCH // 2, _pair, 0)
    plsc.subcore_barrier()

    for k in range(RPT // B):
        pltpu.sync_copy(summ_acc.at[pl.ds(r0 + k * B, B)],
                        summ_out.at[c, pl.ds(r0 + k * B, B)])


_agg = pl.kernel(
    _agg_body,
    out_type=jax.ShapeDtypeStruct((NC, N_PAD, D), jnp.float32),
    mesh=_mesh,
    scratch_types=[
        pltpu.VMEM_SHARED((N_PAD, D), jnp.float32),
        pltpu.VMEM((IC, B), jnp.int32),
        pltpu.VMEM((IC, B), jnp.int32),
        pltpu.VMEM((IC, B), jnp.int32),
        pltpu.VMEM((IC, B), jnp.int32),
        pltpu.VMEM((B, D), jnp.float32),
        pltpu.VMEM((B, D), jnp.float32),
        pltpu.SemaphoreType.DMA,
        pltpu.SemaphoreType.DMA,
        pltpu.SemaphoreType.DMA,
        pltpu.SemaphoreType.DMA,
        pltpu.SemaphoreType.DMA,
        pltpu.SemaphoreType.DMA,
    ],
)


def _cnt_body(dstb, cnt_out, cnt_acc, dstA, dstB, rows0, ones_v,
              isemA, isemB):
    c = lax.axis_index("c")
    s = lax.axis_index("s")
    base = s * NBW
    r0 = s * RPT

    def _fill(i, carry):
        for k in range(D // 16):
            rows0[i, pl.ds(k * 16, 16)] = jnp.zeros((16,), jnp.float32)
            ones_v[i, pl.ds(k * 16, 16)] = jnp.ones((16,), jnp.float32)
        return carry
    lax.fori_loop(0, B, _fill, 0)
    for k in range(RPT // B):
        pltpu.sync_copy(rows0, cnt_acc.at[pl.ds(r0 + k * B, B)])
    pltpu.sync_copy(dstb.at[c, pl.ds(base, IC)], dstA)
    plsc.subcore_barrier()

    def _do_chunk(ch, dstX, dstY, isemY, more):
        for j in range(IC):
            if j == 0:
                def _prefetch():
                    cb = base + (ch + 1) * IC
                    pltpu.async_copy(dstb.at[c, pl.ds(cb, IC)], dstY, isemY)
                if more is True:
                    _prefetch()
                else:
                    pl.when(more)(_prefetch)
            if j == IC - 1:
                def _wait_idx():
                    cb = base + (ch + 1) * IC
                    pltpu.make_async_copy(dstb.at[c, pl.ds(cb, IC)], dstY,
                                          isemY).wait()
                if more is True:
                    _wait_idx()
                else:
                    pl.when(more)(_wait_idx)
            pltpu.sync_copy(ones_v, cnt_acc.at[dstX.at[j]], add=True)

    def _pair(u, carry):
        _do_chunk(2 * u, dstA, dstB, isemB, True)
        _do_chunk(2 * u + 1, dstB, dstA, isemA, u + 1 < NCH // 2)
        return carry
    lax.fori_loop(0, NCH // 2, _pair, 0)
    plsc.subcore_barrier()

    for k in range(RPT // B):
        pltpu.sync_copy(cnt_acc.at[pl.ds(r0 + k * B, B)],
                        cnt_out.at[c, pl.ds(r0 + k * B, B)])


_cnt = pl.kernel(
    _cnt_body,
    out_type=jax.ShapeDtypeStruct((NC, N_PAD, D), jnp.float32),
    mesh=_mesh,
    scratch_types=[
        pltpu.VMEM_SHARED((N_PAD, D), jnp.float32),
        pltpu.VMEM((IC, B), jnp.int32),
        pltpu.VMEM((IC, B), jnp.int32),
        pltpu.VMEM((B, D), jnp.float32),
        pltpu.VMEM((B, D), jnp.float32),
        pltpu.SemaphoreType.DMA,
        pltpu.SemaphoreType.DMA,
    ],
)


BN = 1000
_G = N // BN


def _dense1_body(s01, cnt, xu, xi,
                 Wl_ui, bl_ui, Wr_ui, Wl_iu, bl_iu, Wr_iu, t2_out):
    mean0 = s01[0] / jnp.maximum(cnt[0][:, 0:1], 1.0)
    t2_out[1, :, :] = (jnp.dot(mean0, Wl_ui[...],
                               preferred_element_type=jnp.float32)
                       + bl_ui[...]
                       + jnp.dot(xi[...], Wr_ui[...],
                                 preferred_element_type=jnp.float32))
    mean1 = s01[1] / jnp.maximum(cnt[1][:, 0:1], 1.0)
    t2_out[0, :, :] = (jnp.dot(mean1, Wl_iu[...],
                               preferred_element_type=jnp.float32)
                       + bl_iu[...]
                       + jnp.dot(xu[...], Wr_iu[...],
                                 preferred_element_type=jnp.float32))


def _dense2_body(s01, cnt, t2,
                 Wl_ui, bl_ui, Wr_ui, Wl_iu, bl_iu, Wr_iu,
                 fu, cu, iu, fi, ci, ii,
                 ou_out, oi_out):
    mean0 = s01[0] / jnp.maximum(cnt[0][:, 0:1], 1.0)
    t_i = jnp.tanh(jnp.dot(mean0, Wl_ui[...],
                           preferred_element_type=jnp.float32)
                   + bl_ui[...]
                   + jnp.dot(t2[1], Wr_ui[...],
                             preferred_element_type=jnp.float32))
    oi_out[...] = fi[...] * ci[...] + ii[...] * t_i
    mean1 = s01[1] / jnp.maximum(cnt[1][:, 0:1], 1.0)
    t_u = jnp.tanh(jnp.dot(mean1, Wl_iu[...],
                           preferred_element_type=jnp.float32)
                   + bl_iu[...]
                   + jnp.dot(t2[0], Wr_iu[...],
                             preferred_element_type=jnp.float32))
    ou_out[...] = fu[...] * cu[...] + iu[...] * t_u


def _pad_spec():
    return pl.BlockSpec((NC, BN, D), lambda i: (0, i, 0))


def _row_spec():
    return pl.BlockSpec((BN, D), lambda i: (i, 0))


def _w_spec():
    return pl.BlockSpec((D, D), lambda i: (0, 0))


def _b_spec():
    return pl.BlockSpec((1, D), lambda i: (0, 0))


_dense1 = pl.pallas_call(
    _dense1_body,
    grid=(_G,),
    in_specs=[_pad_spec(), _pad_spec(), _row_spec(), _row_spec(),
              _w_spec(), _b_spec(), _w_spec(), _w_spec(), _b_spec(), _w_spec()],
    out_specs=pl.BlockSpec((NC, BN, D), lambda i: (0, i, 0)),
    out_shape=jax.ShapeDtypeStruct((NC, N, D), jnp.float32),
)

_dense2 = pl.pallas_call(
    _dense2_body,
    grid=(_G,),
    in_specs=[_pad_spec(), _pad_spec(),
              pl.BlockSpec((NC, BN, D), lambda i: (0, i, 0)),
              _w_spec(), _b_spec(), _w_spec(), _w_spec(), _b_spec(), _w_spec(),
              _row_spec(), _row_spec(), _row_spec(),
              _row_spec(), _row_spec(), _row_spec()],
    out_specs=(_row_spec(), _row_spec()),
    out_shape=(jax.ShapeDtypeStruct((N, D), jnp.float32),
               jax.ShapeDtypeStruct((N, D), jnp.float32)),
)


def _prep_idx(ei, src_offset):
    src = ei[0].astype(jnp.int32)
    dst = ei[1].astype(jnp.int32)
    pad = E_PAD - src.shape[0]
    ar = jnp.arange(pad, dtype=jnp.int32)
    src = jnp.concatenate([src, ar % N]) + src_offset
    dst = jnp.concatenate([dst, N + ar % (N_PAD - N)])
    return src.reshape(NBT, B), dst.reshape(NBT, B)


def kernel(x_user, x_item, edge_index_user_item, edge_index_item_user,
           h_user, h_item, c_user, c_item, i_user, i_item, f_user, f_item,
           Wl1_ui, bl1_ui, Wr1_ui, Wl1_iu, bl1_iu, Wr1_iu,
           Wl2_ui, bl2_ui, Wr2_ui, Wl2_iu, bl2_iu, Wr2_iu):
    src_ui, dst_ui = _prep_idx(edge_index_user_item, 0)
    src_iu, dst_iu = _prep_idx(edge_index_item_user, N)
    srcb = jnp.stack([src_ui, src_iu])
    dstb = jnp.stack([dst_ui, dst_iu])

    cnt = _cnt(dstb)
    table1 = jnp.concatenate([x_user, x_item], axis=0)
    summ1 = _agg(table1, srcb, dstb)

    t2 = _dense1(summ1, cnt, x_user, x_item,
                 Wl1_ui, bl1_ui.reshape(1, D), Wr1_ui,
                 Wl1_iu, bl1_iu.reshape(1, D), Wr1_iu)

    summ2 = _agg(t2.reshape(2 * N, D), srcb, dstb)

    out_u, out_i = _dense2(summ2, cnt, t2,
                           Wl2_ui, bl2_ui.reshape(1, D), Wr2_ui,
                           Wl2_iu, bl2_iu.reshape(1, D), Wr2_iu,
                           f_user, c_user, i_user, f_item, c_item, i_item)
    return out_u, out_i


# speedup vs baseline: 1.0178x; 1.0178x over previous
import jax
import jax.numpy as jnp
from jax import lax
from jax.experimental import pallas as pl
from jax.experimental.pallas import tpu as pltpu
from jax.experimental.pallas import tpu_sc as plsc

N = 10000
D = 128
E = 320000

NC = 2
NS = 16
B = 128
NBW = 160
NBT = NS * NBW
E_PAD = NBT * B
N_PAD = 10240
RPT = N_PAD // NS
IC = 16
NCH = NBW // IC

_mesh = plsc.VectorSubcoreMesh(core_axis_name="c", subcore_axis_name="s",
                               num_cores=NC, num_subcores=NS)


def _agg_body(table, srcb, dstb, summ_out, summ_acc,
              srcA, dstA, srcB, dstB, rows0, rows1,
              gsem0, gsem1, isemA, isemB):
    c = lax.axis_index("c")
    s = lax.axis_index("s")
    base = s * NBW
    r0 = s * RPT
    rows = (rows0, rows1)
    gsems = (gsem0, gsem1)

    def _issue_gather(idx_row, buf):
        pltpu.async_copy(table.at[idx_row], rows[buf], gsems[buf])

    def _wait_gather(buf):
        pltpu.make_async_copy(table.at[pl.ds(0, B)], rows[buf],
                              gsems[buf]).wait()

    def _fill_zrow(i, carry):
        for k in range(D // 16):
            z = jnp.zeros((16,), jnp.float32)
            rows0[i, pl.ds(k * 16, 16)] = z
            rows1[i, pl.ds(k * 16, 16)] = z
        return carry
    lax.fori_loop(0, B, _fill_zrow, 0)
    for k in range(RPT // B):
        pltpu.sync_copy(rows0, summ_acc.at[pl.ds(r0 + k * B, B)])
    pltpu.sync_copy(srcb.at[c, pl.ds(base, IC)], srcA)
    pltpu.sync_copy(dstb.at[c, pl.ds(base, IC)], dstA)
    _issue_gather(srcA.at[0], 0)
    plsc.subcore_barrier()

    def _do_chunk(ch, srcX, dstX, srcY, dstY, isemY, more):
        for j in range(IC):
            b = j % 2
            nb = (j + 1) % 2
            if j == 0:
                def _prefetch():
                    cb = base + (ch + 1) * IC
                    pltpu.async_copy(srcb.at[c, pl.ds(cb, IC)], srcY, isemY)
                    pltpu.async_copy(dstb.at[c, pl.ds(cb, IC)], dstY, isemY)
                if more is True:
                    _prefetch()
                else:
                    pl.when(more)(_prefetch)
            if j < IC - 1:
                _issue_gather(srcX.at[j + 1], nb)
            else:
                def _next_gather():
                    cb = base + (ch + 1) * IC
                    pltpu.make_async_copy(srcb.at[c, pl.ds(cb, IC)], srcY,
                                          isemY).wait()
                    pltpu.make_async_copy(dstb.at[c, pl.ds(cb, IC)], dstY,
                                          isemY).wait()
                    _issue_gather(srcY.at[0], nb)
                if more is True:
                    _next_gather()
                else:
                    pl.when(more)(_next_gather)
            _wait_gather(b)
            pltpu.sync_copy(rows[b], summ_acc.at[dstX.at[j]], add=True)

    def _pair(u, carry):
        _do_chunk(2 * u, srcA, dstA, srcB, dstB, isemB, True)
        _do_chunk(2 * u + 1, srcB, dstB, srcA, dstA, isemA, u + 1 < NCH // 2)
        return carry
    lax.fori_loop(0, NCH // 2, _pair, 0)
    plsc.subcore_barrier()

    for k in range(RPT // B):
        pltpu.sync_copy(summ_acc.at[pl.ds(r0 + k * B, B)],
                        summ_out.at[c, pl.ds(r0 + k * B, B)])


_agg = pl.kernel(
    _agg_body,
    out_type=jax.ShapeDtypeStruct((NC, N_PAD, D), jnp.float32),
    mesh=_mesh,
    scratch_types=[
        pltpu.VMEM_SHARED((N_PAD, D), jnp.float32),
        pltpu.VMEM((IC, B), jnp.int32),
        pltpu.VMEM((IC, B), jnp.int32),
        pltpu.VMEM((IC, B), jnp.int32),
        pltpu.VMEM((IC, B), jnp.int32),
        pltpu.VMEM((B, D), jnp.float32),
        pltpu.VMEM((B, D), jnp.float32),
        pltpu.SemaphoreType.DMA,
        pltpu.SemaphoreType.DMA,
        pltpu.SemaphoreType.DMA,
        pltpu.SemaphoreType.DMA,
    ],
)


def _cnt_body(dstb, cnt_out, cnt_acc, dstA, dstB, rows0, ones_v,
              isemA, isemB):
    c = lax.axis_index("c")
    s = lax.axis_index("s")
    base = s * NBW
    r0 = s * RPT

    def _fill(i, carry):
        for k in range(D // 16):
            rows0[i, pl.ds(k * 16, 16)] = jnp.zeros((16,), jnp.float32)
            ones_v[i, pl.ds(k * 16, 16)] = jnp.ones((16,), jnp.float32)
        return carry
    lax.fori_loop(0, B, _fill, 0)
    for k in range(RPT // B):
        pltpu.sync_copy(rows0, cnt_acc.at[pl.ds(r0 + k * B, B)])
    pltpu.sync_copy(dstb.at[c, pl.ds(base, IC)], dstA)
    plsc.subcore_barrier()

    def _do_chunk(ch, dstX, dstY, isemY, more):
        for j in range(IC):
            if j == 0:
                def _prefetch():
                    cb = base + (ch + 1) * IC
                    pltpu.async_copy(dstb.at[c, pl.ds(cb, IC)], dstY, isemY)
                if more is True:
                    _prefetch()
                else:
                    pl.when(more)(_prefetch)
            if j == IC - 1:
                def _wait_idx():
                    cb = base + (ch + 1) * IC
                    pltpu.make_async_copy(dstb.at[c, pl.ds(cb, IC)], dstY,
                                          isemY).wait()
                if more is True:
                    _wait_idx()
                else:
                    pl.when(more)(_wait_idx)
            pltpu.sync_copy(ones_v, cnt_acc.at[dstX.at[j]], add=True)

    def _pair(u, carry):
        _do_chunk(2 * u, dstA, dstB, isemB, True)
        _do_chunk(2 * u + 1, dstB, dstA, isemA, u + 1 < NCH // 2)
        return carry
    lax.fori_loop(0, NCH // 2, _pair, 0)
    plsc.subcore_barrier()

    for k in range(RPT // B):
        pltpu.sync_copy(cnt_acc.at[pl.ds(r0 + k * B, B)],
                        cnt_out.at[c, pl.ds(r0 + k * B, B)])


_cnt = pl.kernel(
    _cnt_body,
    out_type=jax.ShapeDtypeStruct((NC, N_PAD, D), jnp.float32),
    mesh=_mesh,
    scratch_types=[
        pltpu.VMEM_SHARED((N_PAD, D), jnp.float32),
        pltpu.VMEM((IC, B), jnp.int32),
        pltpu.VMEM((IC, B), jnp.int32),
        pltpu.VMEM((B, D), jnp.float32),
        pltpu.VMEM((B, D), jnp.float32),
        pltpu.SemaphoreType.DMA,
        pltpu.SemaphoreType.DMA,
    ],
)


BN = 1000
_G = N // BN


def _dense1_body(s01, cnt, xu, xi,
                 Wl_ui, bl_ui, Wr_ui, Wl_iu, bl_iu, Wr_iu, t2_out):
    mean0 = s01[0] / jnp.maximum(cnt[0][:, 0:1], 1.0)
    t2_out[1, :, :] = (jnp.dot(mean0, Wl_ui[...],
                               preferred_element_type=jnp.float32)
                       + bl_ui[...]
                       + jnp.dot(xi[...], Wr_ui[...],
                                 preferred_element_type=jnp.float32))
    mean1 = s01[1] / jnp.maximum(cnt[1][:, 0:1], 1.0)
    t2_out[0, :, :] = (jnp.dot(mean1, Wl_iu[...],
                               preferred_element_type=jnp.float32)
                       + bl_iu[...]
                       + jnp.dot(xu[...], Wr_iu[...],
                                 preferred_element_type=jnp.float32))


def _dense2_body(s01, cnt, t2,
                 Wl_ui, bl_ui, Wr_ui, Wl_iu, bl_iu, Wr_iu,
                 fu, cu, iu, fi, ci, ii,
                 ou_out, oi_out):
    mean0 = s01[0] / jnp.maximum(cnt[0][:, 0:1], 1.0)
    t_i = jnp.tanh(jnp.dot(mean0, Wl_ui[...],
                           preferred_element_type=jnp.float32)
                   + bl_ui[...]
                   + jnp.dot(t2[1], Wr_ui[...],
                             preferred_element_type=jnp.float32))
    oi_out[...] = fi[...] * ci[...] + ii[...] * t_i
    mean1 = s01[1] / jnp.maximum(cnt[1][:, 0:1], 1.0)
    t_u = jnp.tanh(jnp.dot(mean1, Wl_iu[...],
                           preferred_element_type=jnp.float32)
                   + bl_iu[...]
                   + jnp.dot(t2[0], Wr_iu[...],
                             preferred_element_type=jnp.float32))
    ou_out[...] = fu[...] * cu[...] + iu[...] * t_u


def _pad_spec():
    return pl.BlockSpec((NC, BN, D), lambda i: (0, i, 0))


def _row_spec():
    return pl.BlockSpec((BN, D), lambda i: (i, 0))


def _w_spec():
    return pl.BlockSpec((D, D), lambda i: (0, 0))


def _b_spec():
    return pl.BlockSpec((1, D), lambda i: (0, 0))


_dense1 = pl.pallas_call(
    _dense1_body,
    grid=(_G,),
    in_specs=[_pad_spec(), _pad_spec(), _row_spec(), _row_spec(),
              _w_spec(), _b_spec(), _w_spec(), _w_spec(), _b_spec(), _w_spec()],
    out_specs=pl.BlockSpec((NC, BN, D), lambda i: (0, i, 0)),
    out_shape=jax.ShapeDtypeStruct((NC, N, D), jnp.float32),
)

_dense2 = pl.pallas_call(
    _dense2_body,
    grid=(_G,),
    in_specs=[_pad_spec(), _pad_spec(),
              pl.BlockSpec((NC, BN, D), lambda i: (0, i, 0)),
              _w_spec(), _b_spec(), _w_spec(), _w_spec(), _b_spec(), _w_spec(),
              _row_spec(), _row_spec(), _row_spec(),
              _row_spec(), _row_spec(), _row_spec()],
    out_specs=(_row_spec(), _row_spec()),
    out_shape=(jax.ShapeDtypeStruct((N, D), jnp.float32),
               jax.ShapeDtypeStruct((N, D), jnp.float32)),
)


def _prep_idx(ei, src_offset):
    src = ei[0].astype(jnp.int32)
    dst = ei[1].astype(jnp.int32)
    pad = E_PAD - src.shape[0]
    ar = jnp.arange(pad, dtype=jnp.int32)
    src = jnp.concatenate([src, ar % N]) + src_offset
    dst = jnp.concatenate([dst, N + ar % (N_PAD - N)])
    return src.reshape(NBT, B), dst.reshape(NBT, B)


def kernel(x_user, x_item, edge_index_user_item, edge_index_item_user,
           h_user, h_item, c_user, c_item, i_user, i_item, f_user, f_item,
           Wl1_ui, bl1_ui, Wr1_ui, Wl1_iu, bl1_iu, Wr1_iu,
           Wl2_ui, bl2_ui, Wr2_ui, Wl2_iu, bl2_iu, Wr2_iu):
    src_ui, dst_ui = _prep_idx(edge_index_user_item, 0)
    src_iu, dst_iu = _prep_idx(edge_index_item_user, N)
    srcb = jnp.stack([src_ui, src_iu])
    dstb = jnp.stack([dst_ui, dst_iu])

    cnt = _cnt(dstb)
    table1 = jnp.concatenate([x_user, x_item], axis=0)
    summ1 = _agg(table1, srcb, dstb)

    t2 = _dense1(summ1, cnt, x_user, x_item,
                 Wl1_ui, bl1_ui.reshape(1, D), Wr1_ui,
                 Wl1_iu, bl1_iu.reshape(1, D), Wr1_iu)

    summ2 = _agg(t2.reshape(2 * N, D), srcb, dstb)

    out_u, out_i = _dense2(summ2, cnt, t2,
                           Wl2_ui, bl2_ui.reshape(1, D), Wr2_ui,
                           Wl2_iu, bl2_iu.reshape(1, D), Wr2_iu,
                           f_user, c_user, i_user, f_item, c_item, i_item)
    return out_u, out_i
